# baseline (speedup 1.0000x reference)
import jax
import jax.numpy as jnp
from jax.experimental import pallas as pl
from jax.experimental.pallas import tpu as pltpu

_B, _N, _K = 64, 196, 4096
_DEPTH = 4
_GROUPS = _B // _DEPTH


def _loss_kernel(mask_ref, ps_hbm, pt_hbm, out_ref,
                 ps_buf, pt_buf, ps_sem, pt_sem):
    def _issue(b, d):
        pltpu.make_async_copy(ps_hbm.at[b], ps_buf.at[d], ps_sem.at[d]).start(priority=d % 2)
        pltpu.make_async_copy(pt_hbm.at[b], pt_buf.at[d], pt_sem.at[d]).start(priority=(d + 1) % 2)

    for d in range(_DEPTH):
        _issue(d, d)

    def body(g, carry):
        acc = carry
        for d in range(_DEPTH):
            b = g * _DEPTH + d
            pltpu.make_async_copy(ps_hbm.at[b], ps_buf.at[d], ps_sem.at[d]).wait()
            pltpu.make_async_copy(pt_hbm.at[b], pt_buf.at[d], pt_sem.at[d]).wait()
            ps = ps_buf[d]
            pt = pt_buf[d]
            m = mask_ref[b]
            safe = jnp.where(m > 0.0, ps, jnp.ones_like(ps))
            acc += jnp.sum(pt * jnp.log(safe) * m)

            @pl.when(g + 1 < _GROUPS)
            def _():
                _issue(b + _DEPTH, d)

        return acc

    num = jax.lax.fori_loop(0, _GROUPS, body, jnp.float32(0.0))
    den = jnp.sum(mask_ref[...])
    out_ref[...] = (-num / den).reshape(1, 1)


def kernel(ps, pt, bool_masked_pos):
    maskf = bool_masked_pos.astype(jnp.float32)[..., None]
    out = pl.pallas_call(
        _loss_kernel,
        in_specs=[
            pl.BlockSpec(memory_space=pltpu.VMEM),
            pl.BlockSpec(memory_space=pl.ANY),
            pl.BlockSpec(memory_space=pl.ANY),
        ],
        out_specs=pl.BlockSpec(memory_space=pltpu.VMEM),
        out_shape=jax.ShapeDtypeStruct((1, 1), jnp.float32),
        scratch_shapes=[
            pltpu.VMEM((_DEPTH, _N, _K), jnp.float32),
            pltpu.VMEM((_DEPTH, _N, _K), jnp.float32),
            pltpu.SemaphoreType.DMA((_DEPTH,)),
            pltpu.SemaphoreType.DMA((_DEPTH,)),
        ],
        compiler_params=pltpu.CompilerParams(
            skip_device_barrier=True,
            disable_semaphore_checks=True,
            disable_bounds_checks=True,
        ),
    )(maskf, ps, pt)
    return out[0, 0]


# speedup vs baseline: 1.0156x; 1.0156x over previous
import jax
import jax.numpy as jnp
from jax.experimental import pallas as pl
from jax.experimental.pallas import tpu as pltpu

_B, _N, _K = 64, 196, 4096
_DEPTH = 4
_GROUPS = _B // _DEPTH


def _loss_kernel(mask_ref, ps_hbm, pt_hbm, out_ref,
                 ps_buf, pt_buf, ps_sem, pt_sem):
    def _issue(b, d):
        pltpu.make_async_copy(ps_hbm.at[b], ps_buf.at[d], ps_sem.at[d]).start(priority=d % 2)
        pltpu.make_async_copy(pt_hbm.at[b], pt_buf.at[d], pt_sem.at[d]).start(priority=(d + 1) % 2)

    for d in range(_DEPTH):
        _issue(d, d)

    def body(g, carry):
        acc = carry
        for d in range(_DEPTH):
            b = g * _DEPTH + d
            pltpu.make_async_copy(ps_hbm.at[b], ps_buf.at[d], ps_sem.at[d]).wait()
            pltpu.make_async_copy(pt_hbm.at[b], pt_buf.at[d], pt_sem.at[d]).wait()
            ps = ps_buf[d]
            pt = pt_buf[d]
            m = mask_ref[b][:, None]
            safe = jnp.where(m > 0.0, ps, jnp.ones_like(ps))
            acc += jnp.sum(pt * jnp.log(safe) * m)

            @pl.when(g + 1 < _GROUPS)
            def _():
                _issue(b + _DEPTH, d)

        return acc

    num = jax.lax.fori_loop(0, _GROUPS, body, jnp.float32(0.0))
    den = jnp.sum(mask_ref[...])
    out_ref[...] = (-num / den).reshape(1, 1)


def kernel(ps, pt, bool_masked_pos):
    maskf = bool_masked_pos.astype(jnp.float32)
    out = pl.pallas_call(
        _loss_kernel,
        in_specs=[
            pl.BlockSpec(memory_space=pltpu.VMEM),
            pl.BlockSpec(memory_space=pl.ANY),
            pl.BlockSpec(memory_space=pl.ANY),
        ],
        out_specs=pl.BlockSpec(memory_space=pltpu.VMEM),
        out_shape=jax.ShapeDtypeStruct((1, 1), jnp.float32),
        scratch_shapes=[
            pltpu.VMEM((_DEPTH, _N, _K), jnp.float32),
            pltpu.VMEM((_DEPTH, _N, _K), jnp.float32),
            pltpu.SemaphoreType.DMA((_DEPTH,)),
            pltpu.SemaphoreType.DMA((_DEPTH,)),
        ],
    )(maskf, ps, pt)
    return out[0, 0]


# speedup vs baseline: 3.6508x; 3.5947x over previous
import jax
import jax.numpy as jnp
from jax.experimental import pallas as pl
from jax.experimental.pallas import tpu as pltpu

_B, _N, _K = 64, 196, 4096
_G = 4
_GRID = _N // _G


def _dense_kernel(mask_ref, ps_ref, pt_ref, num_ref, den_ref):
    i = pl.program_id(0)
    ps = ps_ref[...]
    pt = pt_ref[...]
    m = mask_ref[pl.ds(i * _G, _G), :]
    mb = m[..., None]
    safe = jnp.where(mb > 0.0, ps, jnp.ones_like(ps))
    part = jnp.sum(pt * jnp.log(safe) * mb)

    @pl.when(i == 0)
    def _():
        num_ref[...] = jnp.zeros_like(num_ref)
        den_ref[...] = jnp.sum(mask_ref[...]).reshape(1, 1)

    num_ref[...] += (-part).reshape(1, 1)


def kernel(ps, pt, bool_masked_pos):
    ps_t = jnp.transpose(ps, (1, 0, 2))
    pt_t = jnp.transpose(pt, (1, 0, 2))
    mask_t = bool_masked_pos.astype(jnp.float32).T
    num, den = pl.pallas_call(
        _dense_kernel,
        grid=(_GRID,),
        in_specs=[
            pl.BlockSpec((_N, _B), lambda i: (0, 0)),
            pl.BlockSpec((_G, _B, _K), lambda i: (i, 0, 0)),
            pl.BlockSpec((_G, _B, _K), lambda i: (i, 0, 0)),
        ],
        out_specs=[
            pl.BlockSpec((1, 1), lambda i: (0, 0)),
            pl.BlockSpec((1, 1), lambda i: (0, 0)),
        ],
        out_shape=[
            jax.ShapeDtypeStruct((1, 1), jnp.float32),
            jax.ShapeDtypeStruct((1, 1), jnp.float32),
        ],
        compiler_params=pltpu.CompilerParams(
            dimension_semantics=("arbitrary",),
        ),
    )(mask_t, ps_t, pt_t)
    return num[0, 0] / den[0, 0]
